# baseline (speedup 1.0000x reference)
import functools

import jax
import jax.numpy as jnp
from jax import lax
from jax.experimental import pallas as pl
from jax.experimental.pallas import tpu as pltpu
from jax.experimental.pallas import tpu_sc as plsc

N_NODES_ = 16384
D_ = 1024
N_OCC_ = 8192

_info = plsc.get_sparse_core_info()
_NC, _NS = _info.num_cores, _info.num_subcores
_NW = _NC * _NS
_BPW = N_OCC_ // _NW
_CHUNK = 32
_NCHUNK = _BPW // _CHUNK
_NBUF = 3


def _gather_body(table_hbm, idx_hbm, out_hbm, idx_v, bufs, gsems, ssems):
    wid = lax.axis_index("s") * _NC + lax.axis_index("c")
    base = wid * _BPW
    pltpu.sync_copy(idx_hbm.at[pl.ds(base, _BPW)], idx_v)

    gops = [None] * _NBUF
    sops = [None] * _NBUF
    for b in range(_NBUF - 1):
        gops[b] = pltpu.async_copy(
            table_hbm.at[idx_v.at[pl.ds(b * _CHUNK, _CHUNK)]], bufs[b],
            gsems[b])
    for c in range(_NCHUNK):
        b = c % _NBUF
        gops[b].wait()
        sops[b] = pltpu.async_copy(
            bufs[b], out_hbm.at[pl.ds(base + c * _CHUNK, _CHUNK)], ssems[b])
        nxt = c + _NBUF - 1
        if nxt < _NCHUNK:
            bn = nxt % _NBUF
            if sops[bn] is not None:
                sops[bn].wait()
            gops[bn] = pltpu.async_copy(
                table_hbm.at[idx_v.at[pl.ds(nxt * _CHUNK, _CHUNK)]], bufs[bn],
                gsems[bn])
    for b in range(_NBUF):
        if sops[b] is not None:
            sops[b].wait()


def _body(table_hbm, idx_hbm, out_hbm, idx_v, *scratch):
    bufs = list(scratch[:_NBUF])
    gsems = list(scratch[_NBUF:2 * _NBUF])
    ssems = list(scratch[2 * _NBUF:3 * _NBUF])
    _gather_body(table_hbm, idx_hbm, out_hbm, idx_v, bufs, gsems, ssems)


@jax.jit
def _gather(table, idx):
    mesh = plsc.VectorSubcoreMesh(core_axis_name="c", subcore_axis_name="s")
    run = pl.kernel(
        _body,
        mesh=mesh,
        out_type=jax.ShapeDtypeStruct((N_OCC_, D_), jnp.float32),
        scratch_types=(
            [pltpu.VMEM((_BPW,), jnp.int32)]
            + [pltpu.VMEM((_CHUNK, D_), jnp.float32)] * _NBUF
            + [pltpu.SemaphoreType.DMA] * (2 * _NBUF)
        ),
    )
    return run(table, idx)


def kernel(ast_nodes_encodings, ast_nodes_with_symbol_leaf_nodes_indices,
           ast_nodes_with_symbol_leaf_symbol_idx):
    out = _gather(ast_nodes_encodings, ast_nodes_with_symbol_leaf_nodes_indices)
    return (out, ast_nodes_with_symbol_leaf_symbol_idx)


# speedup vs baseline: 1.0320x; 1.0320x over previous
import functools

import jax
import jax.numpy as jnp
from jax import lax
from jax.experimental import pallas as pl
from jax.experimental.pallas import tpu as pltpu
from jax.experimental.pallas import tpu_sc as plsc

N_NODES_ = 16384
D_ = 1024
N_OCC_ = 8192

_info = plsc.get_sparse_core_info()
_NC, _NS = _info.num_cores, _info.num_subcores
_NW = _NC * _NS
_BPW = N_OCC_ // _NW
_CHUNK = 16
_NCHUNK = _BPW // _CHUNK
_NBUF = 7


def _gather_body(table_hbm, idx_hbm, out_hbm, idx_v, bufs, gsems, ssems):
    wid = lax.axis_index("s") * _NC + lax.axis_index("c")
    base = wid * _BPW
    pltpu.sync_copy(idx_hbm.at[pl.ds(base, _BPW)], idx_v)

    gops = [None] * _NBUF
    sops = [None] * _NBUF
    for b in range(_NBUF - 1):
        gops[b] = pltpu.async_copy(
            table_hbm.at[idx_v.at[pl.ds(b * _CHUNK, _CHUNK)]], bufs[b],
            gsems[b])
    for c in range(_NCHUNK):
        b = c % _NBUF
        gops[b].wait()
        sops[b] = pltpu.async_copy(
            bufs[b], out_hbm.at[pl.ds(base + c * _CHUNK, _CHUNK)], ssems[b])
        nxt = c + _NBUF - 1
        if nxt < _NCHUNK:
            bn = nxt % _NBUF
            if sops[bn] is not None:
                sops[bn].wait()
            gops[bn] = pltpu.async_copy(
                table_hbm.at[idx_v.at[pl.ds(nxt * _CHUNK, _CHUNK)]], bufs[bn],
                gsems[bn])
    for b in range(_NBUF):
        if sops[b] is not None:
            sops[b].wait()


def _body(table_hbm, idx_hbm, out_hbm, idx_v, *scratch):
    bufs = list(scratch[:_NBUF])
    gsems = list(scratch[_NBUF:2 * _NBUF])
    ssems = list(scratch[2 * _NBUF:3 * _NBUF])
    _gather_body(table_hbm, idx_hbm, out_hbm, idx_v, bufs, gsems, ssems)


@jax.jit
def _gather(table, idx):
    mesh = plsc.VectorSubcoreMesh(core_axis_name="c", subcore_axis_name="s")
    run = pl.kernel(
        _body,
        mesh=mesh,
        out_type=jax.ShapeDtypeStruct((N_OCC_, D_), jnp.float32),
        scratch_types=(
            [pltpu.VMEM((_BPW,), jnp.int32)]
            + [pltpu.VMEM((_CHUNK, D_), jnp.float32)] * _NBUF
            + [pltpu.SemaphoreType.DMA] * (2 * _NBUF)
        ),
    )
    return run(table, idx)


def kernel(ast_nodes_encodings, ast_nodes_with_symbol_leaf_nodes_indices,
           ast_nodes_with_symbol_leaf_symbol_idx):
    out = _gather(ast_nodes_encodings, ast_nodes_with_symbol_leaf_nodes_indices)
    return (out, ast_nodes_with_symbol_leaf_symbol_idx)
